# Initial kernel scaffold; baseline (speedup 1.0000x reference)
import jax
import jax.numpy as jnp
from jax.experimental import pallas as pl


def kernel(X_numerical, X_categorical, tables, W1, b1, g1, beta1, W2, b2, g2, beta2, W3, b3, g3, beta3, W4, b4):
    raise NotImplementedError("write your pallas kernel here")


# speedup vs baseline: 7.3089x; 7.3089x over previous
import jax
import jax.numpy as jnp
from jax.experimental import pallas as pl
from jax.experimental.pallas import tpu as pltpu
from jax.experimental.pallas import tpu_sc as plsc

_EPS = 1e-5
_GW = 128


def _sc_gather(tables_flat, flat_idx):
    n = flat_idx.shape[1]
    d = tables_flat.shape[1]
    mesh = plsc.VectorSubcoreMesh(core_axis_name="core", subcore_axis_name="subcore")

    @pl.kernel(
        out_type=jax.ShapeDtypeStruct((n, d), tables_flat.dtype),
        mesh=mesh,
        compiler_params=pltpu.CompilerParams(use_tc_tiling_on_sc=False),
    )
    def gather_kernel(tab_hbm, idx_hbm, out_hbm):
        def body(i_vmem, o_vmem):
            pltpu.sync_copy(tab_hbm.at[i_vmem.at[0]], o_vmem)

        pltpu.emit_pipeline(
            body,
            grid=(n // _GW,),
            in_specs=[pl.BlockSpec((1, _GW), index_map=lambda i: (0, i))],
            out_specs=[pl.BlockSpec((_GW, d), index_map=lambda i: (i, 0))],
            core_axis_name=("core", "subcore"),
            dimension_semantics=(pltpu.PARALLEL,),
        )(idx_hbm, out_hbm)

    return gather_kernel(tables_flat, flat_idx)


_BLK = 2048


def _layer1(xn, emb, w1n, w1e, b1):
    b, h = xn.shape[0], w1n.shape[1]

    def body(xn_ref, e_ref, wn_ref, we_ref, b_ref, z_ref, s_ref, q_ref):
        z = jnp.dot(xn_ref[...], wn_ref[...], preferred_element_type=jnp.float32)
        z = z + jnp.dot(e_ref[...], we_ref[...], preferred_element_type=jnp.float32)
        z = z + b_ref[...]
        z_ref[...] = z

        @pl.when(pl.program_id(0) == 0)
        def _():
            s_ref[...] = jnp.zeros_like(s_ref)
            q_ref[...] = jnp.zeros_like(q_ref)

        s_ref[...] += jnp.sum(z, axis=0)
        q_ref[...] += jnp.sum(z * z, axis=0)

    return pl.pallas_call(
        body,
        grid=(b // _BLK,),
        in_specs=[
            pl.BlockSpec((_BLK, xn.shape[1]), lambda i: (i, 0)),
            pl.BlockSpec((_BLK, emb.shape[1]), lambda i: (i, 0)),
            pl.BlockSpec(w1n.shape, lambda i: (0, 0)),
            pl.BlockSpec(w1e.shape, lambda i: (0, 0)),
            pl.BlockSpec(b1.shape, lambda i: (0,)),
        ],
        out_specs=[
            pl.BlockSpec((_BLK, h), lambda i: (i, 0)),
            pl.BlockSpec((h,), lambda i: (0,)),
            pl.BlockSpec((h,), lambda i: (0,)),
        ],
        out_shape=[
            jax.ShapeDtypeStruct((b, h), jnp.float32),
            jax.ShapeDtypeStruct((h,), jnp.float32),
            jax.ShapeDtypeStruct((h,), jnp.float32),
        ],
        compiler_params=pltpu.CompilerParams(
            dimension_semantics=("arbitrary",)
        ),
    )(xn, emb, w1n, w1e, b1)


def _bn_relu_matmul(z, s, q, g, beta, w, bias):
    b, h_out = z.shape[0], w.shape[1]

    def body(z_ref, s_ref, q_ref, g_ref, be_ref, w_ref, b_ref, z2_ref, s2_ref, q2_ref):
        mu = s_ref[...] * (1.0 / b)
        var = q_ref[...] * (1.0 / b) - mu * mu
        a = g_ref[...] * jax.lax.rsqrt(var + _EPS)
        c = be_ref[...] - a * mu
        h = jnp.maximum(z_ref[...] * a + c, 0.0)
        z2 = jnp.dot(h, w_ref[...], preferred_element_type=jnp.float32) + b_ref[...]
        z2_ref[...] = z2

        @pl.when(pl.program_id(0) == 0)
        def _():
            s2_ref[...] = jnp.zeros_like(s2_ref)
            q2_ref[...] = jnp.zeros_like(q2_ref)

        s2_ref[...] += jnp.sum(z2, axis=0)
        q2_ref[...] += jnp.sum(z2 * z2, axis=0)

    return pl.pallas_call(
        body,
        grid=(b // _BLK,),
        in_specs=[
            pl.BlockSpec((_BLK, z.shape[1]), lambda i: (i, 0)),
            pl.BlockSpec(s.shape, lambda i: (0,)),
            pl.BlockSpec(q.shape, lambda i: (0,)),
            pl.BlockSpec(g.shape, lambda i: (0,)),
            pl.BlockSpec(beta.shape, lambda i: (0,)),
            pl.BlockSpec(w.shape, lambda i: (0, 0)),
            pl.BlockSpec(bias.shape, lambda i: (0,)),
        ],
        out_specs=[
            pl.BlockSpec((_BLK, h_out), lambda i: (i, 0)),
            pl.BlockSpec((h_out,), lambda i: (0,)),
            pl.BlockSpec((h_out,), lambda i: (0,)),
        ],
        out_shape=[
            jax.ShapeDtypeStruct((b, h_out), jnp.float32),
            jax.ShapeDtypeStruct((h_out,), jnp.float32),
            jax.ShapeDtypeStruct((h_out,), jnp.float32),
        ],
        compiler_params=pltpu.CompilerParams(
            dimension_semantics=("arbitrary",)
        ),
    )(z, s, q, g, beta, w, bias)


def _final(z, s, q, g, beta, w4, b4):
    b = z.shape[0]

    def body(z_ref, s_ref, q_ref, g_ref, be_ref, w_ref, b_ref, o_ref):
        mu = s_ref[...] * (1.0 / b)
        var = q_ref[...] * (1.0 / b) - mu * mu
        a = g_ref[...] * jax.lax.rsqrt(var + _EPS)
        c = be_ref[...] - a * mu
        h = jnp.maximum(z_ref[...] * a + c, 0.0)
        logit = jnp.dot(h, w_ref[...], preferred_element_type=jnp.float32) + b_ref[...]
        o_ref[...] = jax.nn.sigmoid(logit)

    return pl.pallas_call(
        body,
        grid=(b // _BLK,),
        in_specs=[
            pl.BlockSpec((_BLK, z.shape[1]), lambda i: (i, 0)),
            pl.BlockSpec(s.shape, lambda i: (0,)),
            pl.BlockSpec(q.shape, lambda i: (0,)),
            pl.BlockSpec(g.shape, lambda i: (0,)),
            pl.BlockSpec(beta.shape, lambda i: (0,)),
            pl.BlockSpec(w4.shape, lambda i: (0, 0)),
            pl.BlockSpec(b4.shape, lambda i: (0,)),
        ],
        out_specs=pl.BlockSpec((_BLK, 1), lambda i: (i, 0)),
        out_shape=jax.ShapeDtypeStruct((b, 1), jnp.float32),
        compiler_params=pltpu.CompilerParams(
            dimension_semantics=("arbitrary",)
        ),
    )(z, s, q, g, beta, w4, b4)


def kernel(X_numerical, X_categorical, tables, W1, b1, g1, beta1, W2, b2, g2, beta2, W3, b3, g3, beta3, W4, b4):
    f, v, d = tables.shape
    b = X_numerical.shape[0]
    nnum = X_numerical.shape[1]

    tables_flat = tables.reshape(f * v, d)
    flat_idx = (X_categorical.astype(jnp.int32)
                + (jnp.arange(f, dtype=jnp.int32) * v)[None, :]).reshape(1, b * f)

    emb_rows = _sc_gather(tables_flat, flat_idx)
    emb = emb_rows.reshape(b, f * d)

    w1n, w1e = W1[:nnum], W1[nnum:]
    z1, s1, q1 = _layer1(X_numerical, emb, w1n, w1e, b1)
    z2, s2, q2 = _bn_relu_matmul(z1, s1, q1, g1, beta1, W2, b2)
    z3, s3, q3 = _bn_relu_matmul(z2, s2, q2, g2, beta2, W3, b3)
    out = _final(z3, s3, q3, g3, beta3, W4, b4)
    return out.reshape(b)
